# baseline (speedup 1.0000x reference)
import functools

import jax
import jax.numpy as jnp
from jax import lax
from jax.experimental import pallas as pl
from jax.experimental.pallas import tpu as pltpu
from jax.experimental.pallas import tpu_sc as plsc

_VOCAB = 50257
_B = 1024
_S = 1024
_D = 16

_NC = 2
_NS = 16
_NW = _NC * _NS
_ROWS_PER_W = _B // _NW
_IDX_MINOR = 128
_IDX_MAJOR = _S // _IDX_MINOR
_NBUF = 3

_TS = _S // 128
_ROW_ELEMS = _S * _D


def _emb_body(x_hbm, emb_hbm, pos_hbm, out_hbm, ibuf, rbuf, sbuf, pos_v,
              isems, gsems, osems):
    wid = lax.axis_index("s") * _NC + lax.axis_index("c")
    base = wid * _ROWS_PER_W

    pltpu.sync_copy(pos_hbm, pos_v)

    lane = lax.broadcasted_iota(jnp.int32, (16,), 0)
    idx_const = (lane // 8) * (8 * _S) + (lane % 8) * 128

    idx_d, g_d, o_d = {}, {}, {}

    def fire_idx(c):
        n = c % _NBUF
        idx_d[c] = pltpu.async_copy(
            x_hbm.at[base + c],
            ibuf.at[pl.ds(n * _S, _S)],
            isems[n],
        )

    def fire_gathers(c):
        n = c % _NBUF
        g_d[c] = [
            pltpu.async_copy(
                emb_hbm.at[ibuf.at[pl.ds(n * _S + j * _IDX_MINOR, _IDX_MINOR)]],
                rbuf.at[n].at[pl.ds(j * _IDX_MINOR, _IDX_MINOR)],
                gsems[n],
            )
            for j in range(_IDX_MAJOR)
        ]

    def fire_out(c):
        n = c % _NBUF
        o_d[c] = pltpu.async_copy(sbuf.at[n], out_hbm.at[base + c], osems[n])

    for c in range(_NBUF):
        fire_idx(c)
    for c in range(2):
        idx_d[c].wait()
        fire_gathers(c)

    for c in range(_ROWS_PER_W):
        for g in g_d[c]:
            g.wait()
        if c + _NBUF < _ROWS_PER_W:
            fire_idx(c + _NBUF)
        if c + 2 < _ROWS_PER_W:
            idx_d[c + 2].wait()
            fire_gathers(c + 2)
        if c - _NBUF >= 0:
            o_d[c - _NBUF].wait()

        rb = rbuf.at[c % _NBUF]
        sb = sbuf.at[c % _NBUF]

        def add_t(s, acc, rb=rb, sb=sb):
            v = rb[s, :] + pos_v[s, :]
            base = (s >> 7) * 1024 + (s & 127)
            plsc.store_scatter(sb, [idx_const + base], v)
            return acc

        lax.fori_loop(0, _S, add_t, 0, unroll=8)

        fire_out(c)

    for c in range(_ROWS_PER_W - _NBUF, _ROWS_PER_W):
        o_d[c].wait()


@functools.partial(
    pl.kernel,
    out_type=jax.ShapeDtypeStruct((_B, _ROW_ELEMS), jnp.float32),
    mesh=plsc.VectorSubcoreMesh(core_axis_name="c", subcore_axis_name="s"),
    scratch_types=[
        pltpu.VMEM((_NBUF * _S,), jnp.int32),
        pltpu.VMEM((_NBUF, _S, _D), jnp.float32),
        pltpu.VMEM((_NBUF, _ROW_ELEMS), jnp.float32),
        pltpu.VMEM((_S, _D), jnp.float32),
        [pltpu.SemaphoreType.DMA] * _NBUF,
        [pltpu.SemaphoreType.DMA] * _NBUF,
        [pltpu.SemaphoreType.DMA] * _NBUF,
    ],
    compiler_params=pltpu.CompilerParams(
        use_tc_tiling_on_sc=False, needs_layout_passes=False
    ),
)
def _emb_kernel(x_hbm, emb_hbm, pos_hbm, out_hbm, ibuf, rbuf, sbuf, pos_v,
                isems, gsems, osems):
    _emb_body(x_hbm, emb_hbm, pos_hbm, out_hbm, ibuf, rbuf, sbuf, pos_v,
              isems, gsems, osems)


def kernel(x, token_emb, token_pos):
    out = _emb_kernel(x.astype(jnp.int32), token_emb, token_pos)
    return (
        out.reshape(_B, _D // 8, _TS, 8, 128)
        .transpose(0, 2, 4, 1, 3)
        .reshape(_B, _S, _D)
    )


# speedup vs baseline: 1.1930x; 1.1930x over previous
import functools

import jax
import jax.numpy as jnp
from jax import lax
from jax.experimental import pallas as pl
from jax.experimental.pallas import tpu as pltpu
from jax.experimental.pallas import tpu_sc as plsc

_VOCAB = 50257
_B = 1024
_S = 1024
_D = 16

_NC = 2
_NS = 16
_NW = _NC * _NS
_ROWS_PER_W = _B // _NW
_IDX_MINOR = 128
_IDX_MAJOR = _S // _IDX_MINOR
_NBUF = 3

_TS = _S // 128
_ROW_ELEMS = _S * _D


def _emb_body(x_hbm, emb_hbm, pos_hbm, out_hbm, ibuf, rbuf, sbuf, pos_v,
              tbuf, isems, gsems, osems):
    wid = lax.axis_index("s") * _NC + lax.axis_index("c")
    base = wid * _ROWS_PER_W

    pltpu.sync_copy(pos_hbm, pos_v)

    lane = lax.broadcasted_iota(jnp.int32, (16,), 0)
    idx17 = lane * 17

    idx_d, g_d, o_d = {}, {}, {}

    def fire_idx(c):
        n = c % _NBUF
        idx_d[c] = pltpu.async_copy(
            x_hbm.at[base + c],
            ibuf.at[pl.ds(n * _S, _S)],
            isems[n],
        )

    def fire_gathers(c):
        n = c % _NBUF
        g_d[c] = [
            pltpu.async_copy(
                emb_hbm.at[ibuf.at[pl.ds(n * _S + j * _IDX_MINOR, _IDX_MINOR)]],
                rbuf.at[n].at[pl.ds(j * _IDX_MINOR, _IDX_MINOR)],
                gsems[n],
            )
            for j in range(_IDX_MAJOR)
        ]

    def fire_out(c):
        n = c % _NBUF
        o_d[c] = pltpu.async_copy(sbuf.at[n], out_hbm.at[base + c], osems[n])

    for c in range(_NBUF):
        fire_idx(c)
    for c in range(2):
        idx_d[c].wait()
        fire_gathers(c)

    for c in range(_ROWS_PER_W):
        for g in g_d[c]:
            g.wait()
        if c + _NBUF < _ROWS_PER_W:
            fire_idx(c + _NBUF)
        if c + 2 < _ROWS_PER_W:
            idx_d[c + 2].wait()
            fire_gathers(c + 2)
        if c - _NBUF >= 0:
            o_d[c - _NBUF].wait()

        rb = rbuf.at[c % _NBUF]
        sb = sbuf.at[c % _NBUF]

        def xpose_block(k, acc, rb=rb, sb=sb):
            s0 = k * 16
            dyn = (k >> 3) * 1024 + (k & 7) * 16
            for j in range(16):
                tbuf[pl.ds(j * 17, 16)] = rb[s0 + j, :] + pos_v[s0 + j, :]
            for d in range(16):
                w = plsc.load_gather(tbuf, [idx17 + d])
                sb[pl.ds(dyn + (d // 8) * (8 * _S) + (d % 8) * 128, 16)] = w
            return acc

        lax.fori_loop(0, _S // 16, xpose_block, 0)

        fire_out(c)

    for c in range(_ROWS_PER_W - _NBUF, _ROWS_PER_W):
        o_d[c].wait()


@functools.partial(
    pl.kernel,
    out_type=jax.ShapeDtypeStruct((_B, _ROW_ELEMS), jnp.float32),
    mesh=plsc.VectorSubcoreMesh(core_axis_name="c", subcore_axis_name="s"),
    scratch_types=[
        pltpu.VMEM((_NBUF * _S,), jnp.int32),
        pltpu.VMEM((_NBUF, _S, _D), jnp.float32),
        pltpu.VMEM((_NBUF, _ROW_ELEMS), jnp.float32),
        pltpu.VMEM((_S, _D), jnp.float32),
        pltpu.VMEM((280,), jnp.float32),
        [pltpu.SemaphoreType.DMA] * _NBUF,
        [pltpu.SemaphoreType.DMA] * _NBUF,
        [pltpu.SemaphoreType.DMA] * _NBUF,
    ],
    compiler_params=pltpu.CompilerParams(
        use_tc_tiling_on_sc=False, needs_layout_passes=False
    ),
)
def _emb_kernel(x_hbm, emb_hbm, pos_hbm, out_hbm, ibuf, rbuf, sbuf, pos_v,
                tbuf, isems, gsems, osems):
    _emb_body(x_hbm, emb_hbm, pos_hbm, out_hbm, ibuf, rbuf, sbuf, pos_v,
              tbuf, isems, gsems, osems)


def kernel(x, token_emb, token_pos):
    out = _emb_kernel(x.astype(jnp.int32), token_emb, token_pos)
    return (
        out.reshape(_B, _D // 8, _TS, 8, 128)
        .transpose(0, 2, 4, 1, 3)
        .reshape(_B, _S, _D)
    )


# speedup vs baseline: 1.2881x; 1.0797x over previous
import functools

import jax
import jax.numpy as jnp
from jax import lax
from jax.experimental import pallas as pl
from jax.experimental.pallas import tpu as pltpu
from jax.experimental.pallas import tpu_sc as plsc

_VOCAB = 50257
_B = 1024
_S = 1024
_D = 16

_NC = 2
_NS = 16
_NW = _NC * _NS
_ROWS_PER_W = _B // _NW
_IDX_MINOR = 128
_IDX_MAJOR = _S // _IDX_MINOR

_ROW_ELEMS = _S * _D
_TB = 280


def _emb_body(x_hbm, emb_hbm, pos_hbm, out_hbm, ibuf, rbuf, sbuf, pos_v,
              tbuf, isems, gsems, osems):
    wid = lax.axis_index("s") * _NC + lax.axis_index("c")
    base = wid * _ROWS_PER_W

    pltpu.sync_copy(pos_hbm, pos_v)

    lane = lax.broadcasted_iota(jnp.int32, (16,), 0)
    idx17 = lane * 17

    def fire_idx(sub, c):
        pltpu.async_copy(
            x_hbm.at[base + c], ibuf.at[pl.ds(sub * _S, _S)], isems[sub]
        )

    def wait_idx(sub):
        pltpu.make_async_copy(
            x_hbm.at[base], ibuf.at[pl.ds(sub * _S, _S)], isems[sub]
        ).wait()

    def fire_gathers(sub):
        for j in range(_IDX_MAJOR):
            pltpu.async_copy(
                emb_hbm.at[
                    ibuf.at[pl.ds(sub * _S + j * _IDX_MINOR, _IDX_MINOR)]
                ],
                rbuf.at[sub].at[pl.ds(j * _IDX_MINOR, _IDX_MINOR)],
                gsems[sub],
            )

    def wait_gathers(sub):
        pltpu.make_async_copy(
            emb_hbm.at[pl.ds(0, _S)], rbuf.at[sub], gsems[sub]
        ).wait()

    def fire_out(sub, c):
        pltpu.async_copy(sbuf.at[sub], out_hbm.at[base + c], osems[sub])

    def wait_out(sub):
        pltpu.make_async_copy(
            out_hbm.at[base], sbuf.at[sub], osems[sub]
        ).wait()

    def xpose(sub):
        rb = rbuf.at[sub]
        sb = sbuf.at[sub]

        def block(k2, acc):
            for h in range(2):
                k = k2 * 2 + h
                s0 = k * 16
                for j in range(16):
                    tbuf[pl.ds(h * _TB + j * 17, 16)] = (
                        rb[s0 + j, :] + pos_v[s0 + j, :]
                    )
            for h in range(2):
                k = k2 * 2 + h
                dyn = (k >> 3) * 1024 + (k & 7) * 16
                for d in range(16):
                    w = plsc.load_gather(tbuf, [idx17 + (h * _TB + d)])
                    sb[pl.ds(dyn + (d // 8) * (8 * _S) + (d % 8) * 128, 16)] = w
            return acc

        lax.fori_loop(0, _S // 32, block, 0)

    fire_idx(0, 0)
    fire_idx(1, 1)
    wait_idx(0)
    fire_gathers(0)
    wait_idx(1)
    fire_gathers(1)

    def pair(m, acc):
        c0 = m * 2
        for sub in range(2):
            c = c0 + sub
            wait_gathers(sub)

            @pl.when(c + 2 < _ROWS_PER_W)
            def _():
                fire_idx(sub, c + 2)

            @pl.when(c >= 2)
            def _():
                wait_out(sub)

            xpose(sub)
            fire_out(sub, c)

            @pl.when(c + 2 < _ROWS_PER_W)
            def _():
                wait_idx(sub)
                fire_gathers(sub)

        return acc

    lax.fori_loop(0, _ROWS_PER_W // 2, pair, 0)

    wait_out(0)
    wait_out(1)


@functools.partial(
    pl.kernel,
    out_type=jax.ShapeDtypeStruct((_B, _ROW_ELEMS), jnp.float32),
    mesh=plsc.VectorSubcoreMesh(core_axis_name="c", subcore_axis_name="s"),
    scratch_types=[
        pltpu.VMEM((2 * _S,), jnp.int32),
        pltpu.VMEM((2, _S, _D), jnp.float32),
        pltpu.VMEM((2, _ROW_ELEMS), jnp.float32),
        pltpu.VMEM((_S, _D), jnp.float32),
        pltpu.VMEM((2 * _TB,), jnp.float32),
        [pltpu.SemaphoreType.DMA] * 2,
        [pltpu.SemaphoreType.DMA] * 2,
        [pltpu.SemaphoreType.DMA] * 2,
    ],
    compiler_params=pltpu.CompilerParams(
        use_tc_tiling_on_sc=False, needs_layout_passes=False
    ),
)
def _emb_kernel(x_hbm, emb_hbm, pos_hbm, out_hbm, ibuf, rbuf, sbuf, pos_v,
                tbuf, isems, gsems, osems):
    _emb_body(x_hbm, emb_hbm, pos_hbm, out_hbm, ibuf, rbuf, sbuf, pos_v,
              tbuf, isems, gsems, osems)


def kernel(x, token_emb, token_pos):
    out = _emb_kernel(x.astype(jnp.int32), token_emb, token_pos)
    return (
        out.reshape(_B, _D // 8, _S // 128, 8, 128)
        .transpose(0, 2, 4, 1, 3)
        .reshape(_B, _S, _D)
    )


# speedup vs baseline: 2.8576x; 2.2184x over previous
import functools

import jax
import jax.numpy as jnp
from jax import lax
from jax.experimental import pallas as pl
from jax.experimental.pallas import tpu as pltpu
from jax.experimental.pallas import tpu_sc as plsc

_VOCAB = 50257
_B = 1024
_S = 1024
_D = 16

_NC = 2
_NS = 16
_NW = _NC * _NS
_ROWS_PER_W = _B // _NW
_IDX_MINOR = 128
_IDX_MAJOR = _S // _IDX_MINOR

_ROW_ELEMS = _S * _D
_TB = 280


def _emb_body(x_hbm, emb_hbm, pos_hbm, out_hbm, ibuf, rbuf, sbuf, pos_v,
              tbuf, isems, gsems, osems):
    wid = lax.axis_index("s") * _NC + lax.axis_index("c")
    base = wid * _ROWS_PER_W

    pltpu.sync_copy(pos_hbm, pos_v)

    lane = lax.broadcasted_iota(jnp.int32, (16,), 0)
    idx17 = lane * 17

    def fire_idx(sub, c):
        pltpu.async_copy(
            x_hbm.at[base + c], ibuf.at[pl.ds(sub * _S, _S)], isems[sub]
        )

    def wait_idx(sub):
        pltpu.make_async_copy(
            x_hbm.at[base], ibuf.at[pl.ds(sub * _S, _S)], isems[sub]
        ).wait()

    def fire_gathers(sub):
        for j in range(_IDX_MAJOR):
            pltpu.async_copy(
                emb_hbm.at[
                    ibuf.at[pl.ds(sub * _S + j * _IDX_MINOR, _IDX_MINOR)]
                ],
                rbuf.at[sub].at[pl.ds(j * _IDX_MINOR, _IDX_MINOR)],
                gsems[sub],
            )

    def wait_gathers(sub):
        pltpu.make_async_copy(
            emb_hbm.at[pl.ds(0, _S)], rbuf.at[sub], gsems[sub]
        ).wait()

    def fire_out(sub, c):
        pltpu.async_copy(sbuf.at[sub], out_hbm.at[base + c], osems[sub])

    def wait_out(sub):
        pltpu.make_async_copy(
            out_hbm.at[base], sbuf.at[sub], osems[sub]
        ).wait()

    def xpose(sub):
        rb = rbuf.at[sub]
        sb = sbuf.at[sub]

        def block(k, acc):
            s0 = k * 16
            sums = [rb[s0 + j, :] + pos_v[s0 + j, :] for j in range(16)]
            for j in range(16):
                tbuf[pl.ds(j * 17, 16)] = sums[j]
            dyn = (k >> 3) * 1024 + (k & 7) * 16
            cols = [plsc.load_gather(tbuf, [idx17 + d]) for d in range(16)]
            for d in range(16):
                sb[pl.ds(dyn + (d // 8) * (8 * _S) + (d % 8) * 128, 16)] = (
                    cols[d]
                )
            return acc

        lax.fori_loop(0, _S // 16, block, 0)

    fire_idx(0, 0)
    fire_idx(1, 1)
    wait_idx(0)
    fire_gathers(0)
    wait_idx(1)
    fire_gathers(1)

    def pair(m, acc):
        c0 = m * 2
        for sub in range(2):
            c = c0 + sub
            wait_gathers(sub)

            @pl.when(c + 2 < _ROWS_PER_W)
            def _():
                fire_idx(sub, c + 2)

            @pl.when(c >= 2)
            def _():
                wait_out(sub)

            xpose(sub)
            fire_out(sub, c)

            @pl.when(c + 2 < _ROWS_PER_W)
            def _():
                wait_idx(sub)
                fire_gathers(sub)

        return acc

    lax.fori_loop(0, _ROWS_PER_W // 2, pair, 0)

    wait_out(0)
    wait_out(1)


@functools.partial(
    pl.kernel,
    out_type=jax.ShapeDtypeStruct((_B, _ROW_ELEMS), jnp.float32),
    mesh=plsc.VectorSubcoreMesh(core_axis_name="c", subcore_axis_name="s"),
    scratch_types=[
        pltpu.VMEM((2 * _S,), jnp.int32),
        pltpu.VMEM((2, _S, _D), jnp.float32),
        pltpu.VMEM((2, _ROW_ELEMS), jnp.float32),
        pltpu.VMEM((_S, _D), jnp.float32),
        pltpu.VMEM((2 * _TB,), jnp.float32),
        [pltpu.SemaphoreType.DMA] * 2,
        [pltpu.SemaphoreType.DMA] * 2,
        [pltpu.SemaphoreType.DMA] * 2,
    ],
    compiler_params=pltpu.CompilerParams(
        use_tc_tiling_on_sc=False, needs_layout_passes=False
    ),
)
def _emb_kernel(x_hbm, emb_hbm, pos_hbm, out_hbm, ibuf, rbuf, sbuf, pos_v,
                tbuf, isems, gsems, osems):
    _emb_body(x_hbm, emb_hbm, pos_hbm, out_hbm, ibuf, rbuf, sbuf, pos_v,
              tbuf, isems, gsems, osems)


def kernel(x, token_emb, token_pos):
    out = _emb_kernel(x.astype(jnp.int32), token_emb, token_pos)
    return (
        out.reshape(_B, _D // 8, _S // 128, 8, 128)
        .transpose(0, 2, 4, 1, 3)
        .reshape(_B, _S, _D)
    )


# speedup vs baseline: 2.8637x; 1.0021x over previous
import functools

import jax
import jax.numpy as jnp
from jax import lax
from jax.experimental import pallas as pl
from jax.experimental.pallas import tpu as pltpu
from jax.experimental.pallas import tpu_sc as plsc

_VOCAB = 50257
_B = 1024
_S = 1024
_D = 16

_NC = 2
_NS = 16
_NW = _NC * _NS
_ROWS_PER_W = _B // _NW
_IDX_MINOR = 1024
_IDX_MAJOR = _S // _IDX_MINOR

_ROW_ELEMS = _S * _D
_TB = 280


def _emb_body(x_hbm, emb_hbm, pos_hbm, out_hbm, ibuf, rbuf, sbuf, pos_v,
              tbuf, isems, gsems, osems):
    wid = lax.axis_index("s") * _NC + lax.axis_index("c")
    base = wid * _ROWS_PER_W

    pltpu.sync_copy(pos_hbm, pos_v)

    lane = lax.broadcasted_iota(jnp.int32, (16,), 0)
    idx17 = lane * 17

    def fire_idx(sub, c):
        pltpu.async_copy(
            x_hbm.at[base + c], ibuf.at[pl.ds(sub * _S, _S)], isems[sub]
        )

    def wait_idx(sub):
        pltpu.make_async_copy(
            x_hbm.at[base], ibuf.at[pl.ds(sub * _S, _S)], isems[sub]
        ).wait()

    def fire_gathers(sub):
        for j in range(_IDX_MAJOR):
            pltpu.async_copy(
                emb_hbm.at[
                    ibuf.at[pl.ds(sub * _S + j * _IDX_MINOR, _IDX_MINOR)]
                ],
                rbuf.at[sub].at[pl.ds(j * _IDX_MINOR, _IDX_MINOR)],
                gsems[sub],
            )

    def wait_gathers(sub):
        pltpu.make_async_copy(
            emb_hbm.at[pl.ds(0, _S)], rbuf.at[sub], gsems[sub]
        ).wait()

    def fire_out(sub, c):
        pltpu.async_copy(sbuf.at[sub], out_hbm.at[base + c], osems[sub])

    def wait_out(sub):
        pltpu.make_async_copy(
            out_hbm.at[base], sbuf.at[sub], osems[sub]
        ).wait()

    def xpose(sub):
        rb = rbuf.at[sub]
        sb = sbuf.at[sub]

        def block(k, acc):
            s0 = k * 16
            sums = [rb[s0 + j, :] + pos_v[s0 + j, :] for j in range(16)]
            for j in range(16):
                tbuf[pl.ds(j * 17, 16)] = sums[j]
            dyn = (k >> 3) * 1024 + (k & 7) * 16
            cols = [plsc.load_gather(tbuf, [idx17 + d]) for d in range(16)]
            for d in range(16):
                sb[pl.ds(dyn + (d // 8) * (8 * _S) + (d % 8) * 128, 16)] = (
                    cols[d]
                )
            return acc

        lax.fori_loop(0, _S // 16, block, 0)

    fire_idx(0, 0)
    fire_idx(1, 1)
    wait_idx(0)
    fire_gathers(0)
    wait_idx(1)
    fire_gathers(1)

    def pair(m, acc):
        c0 = m * 2
        for sub in range(2):
            c = c0 + sub
            wait_gathers(sub)

            @pl.when(c + 2 < _ROWS_PER_W)
            def _():
                fire_idx(sub, c + 2)

            @pl.when(c >= 2)
            def _():
                wait_out(sub)

            xpose(sub)
            fire_out(sub, c)

            @pl.when(c + 2 < _ROWS_PER_W)
            def _():
                wait_idx(sub)
                fire_gathers(sub)

        return acc

    lax.fori_loop(0, _ROWS_PER_W // 2, pair, 0)

    wait_out(0)
    wait_out(1)


@functools.partial(
    pl.kernel,
    out_type=jax.ShapeDtypeStruct((_B, _ROW_ELEMS), jnp.float32),
    mesh=plsc.VectorSubcoreMesh(core_axis_name="c", subcore_axis_name="s"),
    scratch_types=[
        pltpu.VMEM((2 * _S,), jnp.int32),
        pltpu.VMEM((2, _S, _D), jnp.float32),
        pltpu.VMEM((2, _ROW_ELEMS), jnp.float32),
        pltpu.VMEM((_S, _D), jnp.float32),
        pltpu.VMEM((2 * _TB,), jnp.float32),
        [pltpu.SemaphoreType.DMA] * 2,
        [pltpu.SemaphoreType.DMA] * 2,
        [pltpu.SemaphoreType.DMA] * 2,
    ],
    compiler_params=pltpu.CompilerParams(
        use_tc_tiling_on_sc=False, needs_layout_passes=False
    ),
)
def _emb_kernel(x_hbm, emb_hbm, pos_hbm, out_hbm, ibuf, rbuf, sbuf, pos_v,
                tbuf, isems, gsems, osems):
    _emb_body(x_hbm, emb_hbm, pos_hbm, out_hbm, ibuf, rbuf, sbuf, pos_v,
              tbuf, isems, gsems, osems)


def kernel(x, token_emb, token_pos):
    out = _emb_kernel(x.astype(jnp.int32), token_emb, token_pos)
    return (
        out.reshape(_B, _D // 8, _S // 128, 8, 128)
        .transpose(0, 2, 4, 1, 3)
        .reshape(_B, _S, _D)
    )
